# Initial kernel scaffold; baseline (speedup 1.0000x reference)
import jax
import jax.numpy as jnp
from jax.experimental import pallas as pl


def kernel(contexts, targets, emb_in, emb_out):
    raise NotImplementedError("write your pallas kernel here")


# speedup vs baseline: 1.1882x; 1.1882x over previous
import jax
import jax.numpy as jnp
from jax import lax
from jax.experimental import pallas as pl
from jax.experimental.pallas import tpu as pltpu
from jax.experimental.pallas import tpu_sc as plsc

B = 4096
D = 64
L = 10
LANES = 16
NW = 32
RPW = B // NW
BLK = 256
NBLK = B // BLK


def _sc_body(ctx_t, tgt_idx, emb_in, emb_out, ctx_out, tgt_out, nz_out,
             idx_v, tidx_v, rows_v, out_v, tgt_v, cnt_v, sem):
    wid = lax.axis_index("s") * 2 + lax.axis_index("c")
    base = wid * RPW

    pltpu.sync_copy(ctx_t.at[:, pl.ds(base, RPW)], idx_v)
    pltpu.sync_copy(tgt_idx.at[pl.ds(base, RPW)], tidx_v)

    copies = []
    for l in range(L):
        copies.append(pltpu.async_copy(emb_in.at[idx_v.at[l]], rows_v.at[l], sem))
    copies.append(pltpu.async_copy(emb_out.at[tidx_v], tgt_v, sem))
    for cp in copies:
        cp.wait()

    one = jnp.float32(1.0)
    zero = jnp.float32(0.0)
    for rc in range(RPW // LANES):
        cnt = jnp.zeros((LANES,), jnp.float32)
        for l in range(L):
            iv = idx_v[l, pl.ds(rc * LANES, LANES)]
            cnt = cnt + jnp.where(iv == 0, one, zero)
        cnt_v[pl.ds(rc * LANES, LANES)] = cnt

    def row_body(r, carry):
        for c in range(D // LANES):
            sl = pl.ds(c * LANES, LANES)
            acc = rows_v[0, r, sl]
            for l in range(1, L):
                acc = acc + rows_v[l, r, sl]
            out_v[r, sl] = acc
        return carry

    lax.fori_loop(0, RPW, row_body, 0)

    pltpu.sync_copy(out_v, ctx_out.at[pl.ds(base, RPW)])
    pltpu.sync_copy(tgt_v, tgt_out.at[pl.ds(base, RPW)])
    pltpu.sync_copy(cnt_v, nz_out.at[pl.ds(base, RPW)])


_sc_gather = pl.kernel(
    _sc_body,
    out_type=(
        jax.ShapeDtypeStruct((B, D), jnp.float32),
        jax.ShapeDtypeStruct((B, D), jnp.float32),
        jax.ShapeDtypeStruct((B,), jnp.float32),
    ),
    mesh=plsc.VectorSubcoreMesh(core_axis_name="c", subcore_axis_name="s"),
    compiler_params=pltpu.CompilerParams(use_tc_tiling_on_sc=False),
    scratch_types=[
        pltpu.VMEM((L, RPW), jnp.int32),
        pltpu.VMEM((RPW,), jnp.int32),
        pltpu.VMEM((L, RPW, D), jnp.float32),
        pltpu.VMEM((RPW, D), jnp.float32),
        pltpu.VMEM((RPW, D), jnp.float32),
        pltpu.VMEM((RPW,), jnp.float32),
        pltpu.SemaphoreType.DMA,
    ],
)


def _tc_body(ctx_ref, tgt_ref, tcls_ref, nz_ref, emb0_ref, out_ref):
    j = pl.program_id(0)
    ctx = ctx_ref[...] - nz_ref[0, 0, :][:, None] * emb0_ref[...]
    ss = jnp.sum(ctx * ctx, axis=1, keepdims=True)
    cn = ctx * lax.rsqrt(jnp.maximum(ss, 1e-24))
    logits = lax.dot_general(
        cn, tgt_ref[...], (((1,), (1,)), ((), ())),
        preferred_element_type=jnp.float32,
    )
    m = jnp.max(logits, axis=1, keepdims=True)
    lse = m[:, 0] + jnp.log(jnp.sum(jnp.exp(logits - m), axis=1))
    t = tcls_ref[0, 0, :]
    cols = lax.broadcasted_iota(jnp.int32, (BLK, B), 1)
    picked = jnp.sum(jnp.where(cols == t[:, None], logits, 0.0), axis=1)
    part = jnp.sum(lse - picked)

    @pl.when(j == 0)
    def _():
        out_ref[0, 0] = 0.0

    acc = out_ref[0, 0] + part
    out_ref[0, 0] = jnp.where(j == NBLK - 1, acc / B, acc)


def kernel(contexts, targets, emb_in, emb_out):
    ctx_t = contexts.astype(jnp.int32).T
    tgt_i = targets.astype(jnp.int32)
    ctx_raw, tgt_rows, nzero = _sc_gather(ctx_t, tgt_i, emb_in, emb_out)
    tcls = tgt_i.reshape(NBLK, 1, BLK)
    nz3 = nzero.reshape(NBLK, 1, BLK)
    emb0 = emb_in[0:1]
    loss = pl.pallas_call(
        _tc_body,
        grid=(NBLK,),
        in_specs=[
            pl.BlockSpec((BLK, D), lambda j: (j, 0)),
            pl.BlockSpec((B, D), lambda j: (0, 0)),
            pl.BlockSpec((1, 1, BLK), lambda j: (j, 0, 0)),
            pl.BlockSpec((1, 1, BLK), lambda j: (j, 0, 0)),
            pl.BlockSpec((1, D), lambda j: (0, 0)),
        ],
        out_specs=pl.BlockSpec(memory_space=pltpu.SMEM),
        out_shape=jax.ShapeDtypeStruct((1, 1), jnp.float32),
    )(ctx_raw, tgt_rows, tcls, nz3, emb0)
    return loss[0, 0]


# speedup vs baseline: 1.6526x; 1.3908x over previous
import jax
import jax.numpy as jnp
from jax import lax
from jax.experimental import pallas as pl
from jax.experimental.pallas import tpu as pltpu
from jax.experimental.pallas import tpu_sc as plsc

B = 4096
D = 64
L = 10
LANES = 16
NW = 32
RPW = B // NW
BLK = 256
NBLK = B // BLK


def _sc_body(ctx_t, tgt_idx, emb_in, emb_out, ctx_out, tgt_out, nz_out,
             idx_v, tidx_v, rows_v, out_v, tgt_v, cnt_v, sem):
    wid = lax.axis_index("s") * 2 + lax.axis_index("c")
    base = wid * RPW

    pltpu.sync_copy(ctx_t.at[:, pl.ds(base, RPW)], idx_v)
    pltpu.sync_copy(tgt_idx.at[pl.ds(base, RPW)], tidx_v)

    copies = []
    for l in range(L):
        copies.append(pltpu.async_copy(emb_in.at[idx_v.at[l]], rows_v.at[l], sem))
    copies.append(pltpu.async_copy(emb_out.at[tidx_v], tgt_v, sem))
    for cp in copies:
        cp.wait()

    one = jnp.float32(1.0)
    zero = jnp.float32(0.0)
    for rc in range(RPW // LANES):
        cnt = jnp.zeros((LANES,), jnp.float32)
        for l in range(L):
            iv = idx_v[l, pl.ds(rc * LANES, LANES)]
            cnt = cnt + jnp.where(iv == 0, one, zero)
        cnt_v[pl.ds(rc * LANES, LANES)] = cnt

    def row_body(r, carry):
        for c in range(D // LANES):
            sl = pl.ds(c * LANES, LANES)
            acc = rows_v[0, r, sl]
            for l in range(1, L):
                acc = acc + rows_v[l, r, sl]
            out_v[r, sl] = acc
        return carry

    lax.fori_loop(0, RPW, row_body, 0)

    pltpu.sync_copy(out_v, ctx_out.at[pl.ds(base, RPW)])
    pltpu.sync_copy(tgt_v, tgt_out.at[pl.ds(base, RPW)])
    pltpu.sync_copy(cnt_v, nz_out.at[pl.ds(base, RPW)])


_sc_gather = pl.kernel(
    _sc_body,
    out_type=(
        jax.ShapeDtypeStruct((B, D), jnp.float32),
        jax.ShapeDtypeStruct((B, D), jnp.float32),
        jax.ShapeDtypeStruct((B,), jnp.float32),
    ),
    mesh=plsc.VectorSubcoreMesh(core_axis_name="c", subcore_axis_name="s"),
    compiler_params=pltpu.CompilerParams(use_tc_tiling_on_sc=False),
    scratch_types=[
        pltpu.VMEM((L, RPW), jnp.int32),
        pltpu.VMEM((RPW,), jnp.int32),
        pltpu.VMEM((L, RPW, D), jnp.float32),
        pltpu.VMEM((RPW, D), jnp.float32),
        pltpu.VMEM((RPW, D), jnp.float32),
        pltpu.VMEM((RPW,), jnp.float32),
        pltpu.SemaphoreType.DMA,
    ],
)


def _tc_body(ctx_ref, tgt_ref, tcls_ref, nz_ref, emb0_ref, out_ref):
    j = pl.program_id(0)
    ctx = ctx_ref[...] - nz_ref[0, 0, :][:, None] * emb0_ref[...]
    ss = jnp.sum(ctx * ctx, axis=1, keepdims=True)
    cn = ctx * lax.rsqrt(jnp.maximum(ss, 1e-24))
    logits = lax.dot_general(
        cn, tgt_ref[...], (((1,), (1,)), ((), ())),
        preferred_element_type=jnp.float32,
    )
    m = jnp.max(logits, axis=1, keepdims=True)
    lse = m[:, 0] + jnp.log(jnp.sum(jnp.exp(logits - m), axis=1))
    t = tcls_ref[0, 0, :]
    cols = lax.broadcasted_iota(jnp.int32, (BLK, B), 1)
    picked = jnp.sum(jnp.where(cols == t[:, None], logits, 0.0), axis=1)
    part = jnp.sum(lse - picked)

    @pl.when(j == 0)
    def _():
        out_ref[0, 0] = 0.0

    acc = out_ref[0, 0] + part
    out_ref[0, 0] = jnp.where(j == NBLK - 1, acc / B, acc)


def kernel(contexts, targets, emb_in, emb_out):
    ctx_t = contexts.astype(jnp.int32).T
    tgt_i = targets.astype(jnp.int32)
    ctx_raw, tgt_rows, nzero = _sc_gather(ctx_t, tgt_i, emb_in, emb_out[:B])
    tcls = tgt_i.reshape(NBLK, 1, BLK)
    nz3 = nzero.reshape(NBLK, 1, BLK)
    emb0 = emb_in[0:1]
    loss = pl.pallas_call(
        _tc_body,
        grid=(NBLK,),
        in_specs=[
            pl.BlockSpec((BLK, D), lambda j: (j, 0)),
            pl.BlockSpec((B, D), lambda j: (0, 0)),
            pl.BlockSpec((1, 1, BLK), lambda j: (j, 0, 0)),
            pl.BlockSpec((1, 1, BLK), lambda j: (j, 0, 0)),
            pl.BlockSpec((1, D), lambda j: (0, 0)),
        ],
        out_specs=pl.BlockSpec(memory_space=pltpu.SMEM),
        out_shape=jax.ShapeDtypeStruct((1, 1), jnp.float32),
    )(ctx_raw, tgt_rows, tcls, nz3, emb0)
    return loss[0, 0]


# speedup vs baseline: 2.3934x; 1.4482x over previous
import jax
import jax.numpy as jnp
from jax import lax
from jax.experimental import pallas as pl
from jax.experimental.pallas import tpu as pltpu
from jax.experimental.pallas import tpu_sc as plsc

B = 4096
D = 64
L = 10
V = 100000
LANES = 16
NW = 32
DPW = D // NW
RPW = B // NW
Q = 4
QB = B // Q
BLK = 512
NBLK = B // BLK


def _sc_body(emb_t, ctx_t, tgt_idx, eo_t, ctx_out, tgt_out, tgt2_out, nz_out,
             row_v, idx_v, acc_v, tidx_v, tt_v, erow_v, nz_v, sem, rsem):
    wid = lax.axis_index("s") * 2 + lax.axis_index("c")

    pltpu.sync_copy(ctx_t.at[:, pl.ds(wid * RPW, RPW)], idx_v.at[:, pl.ds(0, RPW)])
    one = jnp.float32(1.0)
    zero = jnp.float32(0.0)
    for rc in range(RPW // LANES):
        cnt = jnp.zeros((LANES,), jnp.float32)
        for l in range(L):
            iv = idx_v[l, pl.ds(rc * LANES, LANES)]
            cnt = cnt + jnp.where(iv == 0, one, zero)
        nz_v[pl.ds(rc * LANES, LANES)] = cnt
    pltpu.sync_copy(nz_v, nz_out.at[pl.ds(wid * RPW, RPW)])

    pltpu.sync_copy(tgt_idx, tidx_v)

    def tt_chunk(ch, carry):
        for u in range(2):
            base = ch * 2 + u
            ii = tidx_v[pl.ds(base * LANES, LANES)]
            tt_v[pl.ds(base * LANES, LANES)] = plsc.load_gather(tidx_v, [ii])
        return carry

    lax.fori_loop(0, B // (2 * LANES), tt_chunk, 0)

    for dd in range(DPW):
        d = wid * DPW + dd

        row_cp = pltpu.async_copy(emb_t.at[d], row_v, rsem)
        pltpu.sync_copy(eo_t.at[d, pl.ds(0, B)], erow_v)

        for src, dst in ((tidx_v, tgt_out), (tt_v, tgt2_out)):
            for q in range(Q):
                def tgt_chunk_q(ch, carry, q=q, src=src):
                    for u in range(2):
                        base = (q * (QB // LANES) + ch * 2 + u) * LANES
                        ii = src[pl.ds(base, LANES)]
                        acc_v[pl.ds((ch * 2 + u) * LANES, LANES)] = (
                            plsc.load_gather(erow_v, [ii]))
                    return carry
                lax.fori_loop(0, QB // (2 * LANES), tgt_chunk_q, 0)
                pltpu.sync_copy(acc_v, dst.at[d, pl.ds(q * QB, QB)])

        row_cp.wait()
        for q in range(Q):
            pltpu.sync_copy(ctx_t.at[:, pl.ds(q * QB, QB)], idx_v)

            def ctx_chunk(ch, carry):
                acc = jnp.zeros((LANES,), jnp.float32)
                for l in range(L):
                    ii = idx_v[l, pl.ds(ch * LANES, LANES)]
                    acc = acc + plsc.load_gather(row_v, [ii])
                acc_v[pl.ds(ch * LANES, LANES)] = acc
                return carry

            lax.fori_loop(0, QB // LANES, ctx_chunk, 0)
            pltpu.sync_copy(acc_v, ctx_out.at[d, pl.ds(q * QB, QB)])


_sc_gather = pl.kernel(
    _sc_body,
    out_type=(
        jax.ShapeDtypeStruct((D, B), jnp.float32),
        jax.ShapeDtypeStruct((D, B), jnp.float32),
        jax.ShapeDtypeStruct((D, B), jnp.float32),
        jax.ShapeDtypeStruct((B,), jnp.float32),
    ),
    mesh=plsc.VectorSubcoreMesh(core_axis_name="c", subcore_axis_name="s"),
    compiler_params=pltpu.CompilerParams(needs_layout_passes=False),
    scratch_types=[
        pltpu.VMEM((V,), jnp.float32),
        pltpu.VMEM((L, QB), jnp.int32),
        pltpu.VMEM((QB,), jnp.float32),
        pltpu.VMEM((B,), jnp.int32),
        pltpu.VMEM((B,), jnp.int32),
        pltpu.VMEM((B,), jnp.float32),
        pltpu.VMEM((RPW,), jnp.float32),
        pltpu.SemaphoreType.DMA,
        pltpu.SemaphoreType.DMA,
    ],
)


def _tc_body(ctx_ref, tgt_ref, tgt2_ref, nz_ref, e0_ref, out_ref):
    j = pl.program_id(0)
    cb = ctx_ref[...] - e0_ref[...] * nz_ref[0, 0, :][None, :]
    ss = jnp.sum(cb * cb, axis=0, keepdims=True)
    cn = cb * lax.rsqrt(jnp.maximum(ss, 1e-24))
    logits = lax.dot_general(
        cn, tgt_ref[...], (((0,), (0,)), ((), ())),
        preferred_element_type=jnp.float32,
    )
    m = jnp.max(logits, axis=1, keepdims=True)
    lse = m[:, 0] + jnp.log(jnp.sum(jnp.exp(logits - m), axis=1))
    picked = jnp.sum(cn * tgt2_ref[...], axis=0)
    part = jnp.sum(lse - picked)

    @pl.when(j == 0)
    def _():
        out_ref[0, 0] = 0.0

    acc = out_ref[0, 0] + part
    out_ref[0, 0] = jnp.where(j == NBLK - 1, acc / B, acc)


def kernel(contexts, targets, emb_in, emb_out):
    ctx_t = contexts.astype(jnp.int32).T
    tgt_i = targets.astype(jnp.int32)
    emb_t = emb_in.T
    eo_t = emb_out.T
    ctx_raw_t, tgt_t, tgt2_t, nzero = _sc_gather(emb_t, ctx_t, tgt_i, eo_t)
    nz3 = nzero.reshape(NBLK, 1, BLK)
    e0 = emb_t[:, 0:1]
    loss = pl.pallas_call(
        _tc_body,
        grid=(NBLK,),
        in_specs=[
            pl.BlockSpec((D, BLK), lambda j: (0, j)),
            pl.BlockSpec((D, B), lambda j: (0, 0)),
            pl.BlockSpec((D, BLK), lambda j: (0, j)),
            pl.BlockSpec((1, 1, BLK), lambda j: (j, 0, 0)),
            pl.BlockSpec((D, 1), lambda j: (0, 0)),
        ],
        out_specs=pl.BlockSpec(memory_space=pltpu.SMEM),
        out_shape=jax.ShapeDtypeStruct((1, 1), jnp.float32),
    )(ctx_raw_t, tgt_t, tgt2_t, nz3, e0)
    return loss[0, 0]
